# Initial kernel scaffold; baseline (speedup 1.0000x reference)
import jax
import jax.numpy as jnp
from jax.experimental import pallas as pl


def kernel(x, edge_index_rel0, edge_index_rel1, edge_index_rel2, W1, b1, W2, b2):
    raise NotImplementedError("write your pallas kernel here")


# speedup vs baseline: 1.7785x; 1.7785x over previous
import functools

import jax
import jax.numpy as jnp
from jax import lax
from jax.experimental import pallas as pl
from jax.experimental.pallas import tpu as pltpu
from jax.experimental.pallas import tpu_sc as plsc

N = 10000
D = 128
E = 100000
R = 3

NC = 2
NS = 16
NW = NC * NS

CHUNK = 128
CPT = 25
EPT = CPT * CHUNK
E_PAD = NW * EPT

ROWS_PER_TILE = 632
N_ROWS = NS * ROWS_PER_TILE
N_DEG = N_ROWS

BN = 400
GRID = N // BN

_f32 = jnp.float32


def _sc_mesh():
    return plsc.VectorSubcoreMesh(core_axis_name="c", subcore_axis_name="s")


@functools.partial(
    pl.kernel,
    out_type=jax.ShapeDtypeStruct((NC * 2 * R * N_DEG,), _f32),
    mesh=_sc_mesh(),
    scratch_types=[
        pltpu.VMEM((CPT, CHUNK), jnp.int32),
        pltpu.VMEM((CHUNK,), _f32),
        pltpu.VMEM((ROWS_PER_TILE,), _f32),
    ] + [pltpu.VMEM_SHARED((N_DEG,), _f32) for _ in range(2 * R)],
)
def _degree_kernel(e0s, e0d, e1s, e1d, e2s, e2d, out,
                   idx, ones, zbuf, a0, a1, a2, a3, a4, a5):
    cid = lax.axis_index("c")
    sid = lax.axis_index("s")
    wid = sid * NC + cid
    accs = [a0, a1, a2, a3, a4, a5]

    def fill_ones(i, c):
        ones[pl.ds(i * 16, 16)] = jnp.ones((16,), _f32)
        return c
    lax.fori_loop(0, CHUNK // 16, fill_ones, 0)

    def fill_zero(i, c):
        zbuf[pl.ds(i * 16, 16)] = jnp.zeros((16,), _f32)
        return c
    lax.fori_loop(0, ROWS_PER_TILE // 16, fill_zero, 0)

    base = sid * ROWS_PER_TILE
    for k in range(2 * R):
        pltpu.sync_copy(zbuf, accs[k].at[pl.ds(base, ROWS_PER_TILE)])
    plsc.subcore_barrier()

    es = [e0s, e0d, e1s, e1d, e2s, e2d]
    for k in range(2 * R):
        def load(c, carry):
            pltpu.sync_copy(es[k].at[pl.ds(wid * EPT + c * CHUNK, CHUNK)],
                            idx.at[c])
            return carry
        lax.fori_loop(0, CPT, load, 0)

        def body(c, carry):
            pltpu.sync_copy(ones, accs[k].at[idx.at[c]], add=True)
            return carry
        lax.fori_loop(0, CPT, body, 0)

    plsc.subcore_barrier()
    for k in range(2 * R):
        pltpu.sync_copy(accs[k].at[pl.ds(base, ROWS_PER_TILE)], zbuf)
        pltpu.sync_copy(
            zbuf,
            out.at[pl.ds((cid * 2 * R + k) * N_DEG + base, ROWS_PER_TILE)])


@functools.partial(
    pl.kernel,
    out_type=jax.ShapeDtypeStruct((NC, N_ROWS, D), _f32),
    mesh=_sc_mesh(),
    scratch_types=[
        pltpu.VMEM((CPT, CHUNK), jnp.int32),
        pltpu.VMEM((CPT, CHUNK), jnp.int32),
        pltpu.VMEM((CHUNK, D), _f32),
        pltpu.VMEM_SHARED((N_ROWS, D), _f32),
        pltpu.SemaphoreType.DMA,
    ],
)
def _agg_kernel(hn, esrc, edst, out, sidx, didx, rows, acc, sem):
    cid = lax.axis_index("c")
    sid = lax.axis_index("s")
    wid = sid * NC + cid

    def zrow(i, c):
        def zcol(j, cc):
            rows[i, pl.ds(j * 16, 16)] = jnp.zeros((16,), _f32)
            return cc
        return lax.fori_loop(0, D // 16, zcol, c)
    lax.fori_loop(0, CHUNK, zrow, 0)

    base = sid * ROWS_PER_TILE
    for off, sz in ((0, 128), (128, 128), (256, 128), (384, 128), (512, 120)):
        pltpu.sync_copy(rows.at[pl.ds(0, sz)], acc.at[pl.ds(base + off, sz)])

    def load(c, carry):
        pltpu.sync_copy(esrc.at[pl.ds(wid * EPT + c * CHUNK, CHUNK)],
                        sidx.at[c])
        pltpu.sync_copy(edst.at[pl.ds(wid * EPT + c * CHUNK, CHUNK)],
                        didx.at[c])
        return carry
    lax.fori_loop(0, CPT, load, 0)
    plsc.subcore_barrier()

    def body(c, carry):
        pltpu.async_copy(hn.at[sidx.at[c]], rows, sem).wait()
        pltpu.sync_copy(rows, acc.at[didx.at[c]], add=True)
        return carry
    lax.fori_loop(0, CPT, body, 0)

    plsc.subcore_barrier()
    for off, sz in ((0, 128), (128, 128), (256, 128), (384, 128), (512, 120)):
        pltpu.sync_copy(acc.at[pl.ds(base + off, sz)], rows.at[pl.ds(0, sz)])
        pltpu.sync_copy(rows.at[pl.ds(0, sz)],
                        out.at[cid, pl.ds(base + off, sz)])


def _dinv_body(dp_ref, out_ref):
    deg = dp_ref[0] + dp_ref[1]
    out_ref[...] = lax.rsqrt(jnp.maximum(deg, 1.0))


_dinv = pl.pallas_call(
    _dinv_body,
    out_shape=jax.ShapeDtypeStruct((N_DEG, 8), _f32),
)


def _scale_body(x_ref, dv_ref, o0, o1, o2):
    xb = x_ref[...]
    outs = (o0, o1, o2)
    for r in range(R):
        outs[r][...] = xb * dv_ref[:, 2 * r:2 * r + 1]


_scale = pl.pallas_call(
    _scale_body,
    grid=(GRID,),
    in_specs=[
        pl.BlockSpec((BN, D), lambda i: (i, 0)),
        pl.BlockSpec((BN, 8), lambda i: (i, 0)),
    ],
    out_specs=[pl.BlockSpec((BN, D), lambda i: (i, 0))] * R,
    out_shape=[jax.ShapeDtypeStruct((N, D), _f32)] * R,
)


def _combine1_body(p0, p1, p2, dv_ref, w_ref, b_ref, o0, o1, o2):
    h = jnp.zeros((BN, D), _f32)
    for r, p in enumerate((p0, p1, p2)):
        agg = (p[0] + p[1]) * dv_ref[:, 2 * r + 1:2 * r + 2]
        y = jnp.dot(agg, w_ref[r], preferred_element_type=_f32) + b_ref[r][None, :]
        h = h + jnp.maximum(y, 0.0)
    outs = (o0, o1, o2)
    for r in range(R):
        outs[r][...] = h * dv_ref[:, 2 * r:2 * r + 1]


_combine1 = pl.pallas_call(
    _combine1_body,
    grid=(GRID,),
    in_specs=[
        pl.BlockSpec((NC, BN, D), lambda i: (0, i, 0)),
        pl.BlockSpec((NC, BN, D), lambda i: (0, i, 0)),
        pl.BlockSpec((NC, BN, D), lambda i: (0, i, 0)),
        pl.BlockSpec((BN, 8), lambda i: (i, 0)),
        pl.BlockSpec((R, D, D), lambda i: (0, 0, 0)),
        pl.BlockSpec((R, D), lambda i: (0, 0)),
    ],
    out_specs=[pl.BlockSpec((BN, D), lambda i: (i, 0))] * R,
    out_shape=[jax.ShapeDtypeStruct((N, D), _f32)] * R,
)


def _combine2_body(p0, p1, p2, dv_ref, w_ref, b_ref, out_ref):
    acc = jnp.zeros((BN, D), _f32)
    for r, p in enumerate((p0, p1, p2)):
        agg = (p[0] + p[1]) * dv_ref[:, 2 * r + 1:2 * r + 2]
        acc = acc + jnp.dot(agg, w_ref[r], preferred_element_type=_f32)
        acc = acc + b_ref[r][None, :]
    out_ref[...] = acc


_combine2 = pl.pallas_call(
    _combine2_body,
    grid=(GRID,),
    in_specs=[
        pl.BlockSpec((NC, BN, D), lambda i: (0, i, 0)),
        pl.BlockSpec((NC, BN, D), lambda i: (0, i, 0)),
        pl.BlockSpec((NC, BN, D), lambda i: (0, i, 0)),
        pl.BlockSpec((BN, 8), lambda i: (i, 0)),
        pl.BlockSpec((R, D, D), lambda i: (0, 0, 0)),
        pl.BlockSpec((R, D), lambda i: (0, 0)),
    ],
    out_specs=pl.BlockSpec((BN, D), lambda i: (i, 0)),
    out_shape=jax.ShapeDtypeStruct((N, D), _f32),
)


def _pad_1d(v, pad_val):
    return jnp.concatenate(
        [v.astype(jnp.int32), jnp.full((E_PAD - E,), pad_val, jnp.int32)])


def kernel(x, edge_index_rel0, edge_index_rel1, edge_index_rel2, W1, b1, W2, b2):
    eis = [edge_index_rel0, edge_index_rel1, edge_index_rel2]
    deg_args = []
    for ei in eis:
        deg_args += [_pad_1d(ei[0], N), _pad_1d(ei[1], N)]
    agg_args = [( _pad_1d(ei[0], 0), _pad_1d(ei[1], N)) for ei in eis]

    degp = _degree_kernel(*deg_args).reshape(NC, 2 * R, N_DEG)
    degp_t = jnp.pad(jnp.transpose(degp, (0, 2, 1)),
                     ((0, 0), (0, 0), (0, 2)))
    dv = _dinv(degp_t)

    hn1 = _scale(x, dv)
    aggs1 = [_agg_kernel(hn1[r], *agg_args[r]) for r in range(R)]
    hn2 = _combine1(aggs1[0], aggs1[1], aggs1[2], dv, W1, b1)
    aggs2 = [_agg_kernel(hn2[r], *agg_args[r]) for r in range(R)]
    return _combine2(aggs2[0], aggs2[1], aggs2[2], dv, W2, b2)
